# speedup vs baseline: 2.6268x; 2.6268x over previous
import functools

import jax
import jax.numpy as jnp
from jax.experimental import pallas as pl
from jax.experimental.pallas import tpu as pltpu


def _update_body(blk, bs, inv_ref, cache_ref, src_ref, out_ref):
    b0 = pl.program_id(0) * blk
    out_ref[...] = cache_ref[...]
    for bb in range(blk):
        for o in range(bs):
            tok = inv_ref[(b0 + bb) * bs + o]

            @pl.when(tok >= 0)
            def _():
                row = src_ref[pl.ds(tok, 1), :, :]
                out_ref[bb, :, o, :] = row.reshape(row.shape[1], row.shape[2])


def _update_cache(cache, src, inv, blk=8):
    nb, h, bs, d = cache.shape
    t = src.shape[0]
    grid = (nb // blk,)
    body = functools.partial(_update_body, blk, bs)
    grid_spec = pltpu.PrefetchScalarGridSpec(
        num_scalar_prefetch=1,
        grid=grid,
        in_specs=[
            pl.BlockSpec((blk, h, bs, d), lambda i, inv_s: (i, 0, 0, 0)),
            pl.BlockSpec((t, h, d), lambda i, inv_s: (0, 0, 0)),
        ],
        out_specs=pl.BlockSpec((blk, h, bs, d), lambda i, inv_s: (i, 0, 0, 0)),
    )
    return pl.pallas_call(
        body,
        grid_spec=grid_spec,
        out_shape=jax.ShapeDtypeStruct(cache.shape, cache.dtype),
        compiler_params=pltpu.CompilerParams(
            vmem_limit_bytes=110 * 1024 * 1024,
        ),
    )(inv, cache, src)


def kernel(k, v, k_cache, v_cache, slot_mapping):
    nb, _, bs, _ = k_cache.shape
    t = k.shape[0]
    valid = slot_mapping >= 0
    slots = jnp.where(valid, slot_mapping, nb * bs)
    inv = jnp.full((nb * bs,), -1, jnp.int32).at[slots].set(
        jnp.arange(t, dtype=jnp.int32), mode="drop"
    )
    k_out = _update_cache(k_cache, k, inv)
    v_out = _update_cache(v_cache, v, inv)
    return (k_out, v_out)
